# Initial kernel scaffold; baseline (speedup 1.0000x reference)
import jax
import jax.numpy as jnp
from jax.experimental import pallas as pl


def kernel(rgb, nir, tir, Wq, bq, Wk, bk, W1, b1, ln_g, ln_b, W2, b2, W3, b3):
    raise NotImplementedError("write your pallas kernel here")


# speedup vs baseline: 3.1466x; 3.1466x over previous
import functools

import jax
import jax.numpy as jnp
from jax.experimental import pallas as pl

_B = 32
_N = 576
_C = 768
_SPARSE_RATIO = 0.6
_SOFT_MASK_TAU = 0.3
_COSINE_TAU = 0.3
_SCALE = _C ** (-0.5)
_K_ORD = 230


def _erf(x):
    return jax.lax.erf(x)


def _gelu_exact(x):
    return 0.5 * x * (1.0 + _erf(x * (2.0 ** -0.5)))


def _globals_kernel(rgb_ref, nir_ref, tir_ref, wq_ref, bq_ref, wk_ref,
                    w1_ref, b1_ref, lng_ref, lnb_ref, w2_ref, b2_ref,
                    w3_ref, b3_ref, v_ref, wm_ref):
    t0 = rgb_ref[0]
    t1 = nir_ref[0]
    t2 = tir_ref[0]
    g0 = jnp.mean(t0, axis=0, keepdims=True)
    g1 = jnp.mean(t1, axis=0, keepdims=True)
    g2 = jnp.mean(t2, axis=0, keepdims=True)
    g = jnp.concatenate([g0, g1, g2], axis=0)

    gn = jnp.sqrt(jnp.sum(g * g, axis=1, keepdims=True))
    gh = g / (gn + 1e-8)

    q = jax.lax.dot_general(g, wq_ref[...],
                            (((1,), (1,)), ((), ())),
                            preferred_element_type=jnp.float32) + bq_ref[...]
    qk = jax.lax.dot_general(q, wk_ref[...],
                             (((1,), (0,)), ((), ())),
                             preferred_element_type=jnp.float32)

    cat = jnp.concatenate([
        jnp.concatenate([g0, g1, g2], axis=1),
        jnp.concatenate([g1, g0, g2], axis=1),
        jnp.concatenate([g2, g0, g1], axis=1),
    ], axis=0)
    h = jax.lax.dot_general(cat, w1_ref[...],
                            (((1,), (1,)), ((), ())),
                            preferred_element_type=jnp.float32) + b1_ref[...]
    mu = jnp.mean(h, axis=1, keepdims=True)
    var = jnp.mean((h - mu) * (h - mu), axis=1, keepdims=True)
    h = (h - mu) / jnp.sqrt(var + 1e-5) * lng_ref[...] + lnb_ref[...]
    h = _gelu_exact(h)
    h = jax.lax.dot_general(h, w2_ref[...],
                            (((1,), (1,)), ((), ())),
                            preferred_element_type=jnp.float32) + b2_ref[...]
    h = _gelu_exact(h)
    logits = jax.lax.dot_general(h, w3_ref[...],
                                 (((1,), (1,)), ((), ())),
                                 preferred_element_type=jnp.float32) + b3_ref[...]
    lmax = jnp.max(logits, axis=1, keepdims=True)
    e = jnp.exp(logits - lmax)
    w = e / jnp.sum(e, axis=1, keepdims=True)

    v_ref[0] = jnp.concatenate([gh, qk, jnp.zeros((2, _C), jnp.float32)], axis=0)
    wm_ref[0] = jnp.concatenate([
        jnp.concatenate([w, jnp.zeros((3, 125), jnp.float32)], axis=1),
        jnp.zeros((5, 128), jnp.float32),
    ], axis=0)


def _score_mask_kernel(rgb_ref, nir_ref, tir_ref, v_ref, wm_ref,
                       out_ref, mask_ref):
    v = v_ref[0]
    wm = wm_ref[0]

    for m, t in ((0, rgb_ref[0]), (1, nir_ref[0]), (2, tir_ref[0])):
        dots = jax.lax.dot_general(t, v, (((1,), (1,)), ((), ())),
                                   preferred_element_type=jnp.float32)
        tnorm = jnp.sqrt(jnp.sum(t * t, axis=1, keepdims=True))
        cos = dots[:, 0:3] / (tnorm + 1e-8)
        logits = dots[:, 3:6] * _SCALE + cos * (1.0 / _COSINE_TAU)
        lmax = jnp.max(logits, axis=0, keepdims=True)
        e = jnp.exp(logits - lmax)
        s = e / jnp.sum(e, axis=0, keepdims=True)
        mean = jnp.mean(s, axis=0, keepdims=True)
        d = s - mean
        sd = jnp.sqrt(jnp.sum(d * d, axis=0, keepdims=True) / (_N - 1)) + 1e-5
        ns = jax.nn.sigmoid(d / sd)
        j0, j1, j2 = ((0, 1, 2), (1, 0, 2), (2, 0, 1))[m]
        score = (ns[:, j0:j0 + 1] * wm[m, 0] + ns[:, j1:j1 + 1] * wm[m, 1]
                 + ns[:, j2:j2 + 1] * wm[m, 2])

        sc_t = jnp.transpose(score)
        c_lt = jnp.sum((sc_t < score).astype(jnp.float32), axis=1, keepdims=True)
        c_le = jnp.sum((sc_t <= score).astype(jnp.float32), axis=1, keepdims=True)
        valid = jnp.logical_and(c_lt <= float(_K_ORD), c_le >= float(_K_ORD + 1))
        thr = jnp.max(jnp.where(valid, score, -jnp.inf))

        mask = jax.nn.sigmoid((score - thr) * (1.0 / _SOFT_MASK_TAU))
        out_ref[m, 0] = t * mask
        mask_ref[m, 0] = jax.nn.sigmoid((sc_t - thr) * (1.0 / _SOFT_MASK_TAU))


@functools.partial(jax.jit)
def kernel(rgb, nir, tir, Wq, bq, Wk, bk, W1, b1, ln_g, ln_b, W2, b2, W3, b3):
    del bk
    bq2 = bq.reshape(1, _C)
    b12 = b1.reshape(1, 256)
    lng2 = ln_g.reshape(1, 256)
    lnb2 = ln_b.reshape(1, 256)
    b22 = b2.reshape(1, 64)
    b32 = b3.reshape(1, 3)

    tok_spec = pl.BlockSpec((1, _N, _C), lambda b: (b, 0, 0))

    def const_spec(shape):
        nd = len(shape)
        return pl.BlockSpec(shape, lambda b, _nd=nd: (0,) * _nd)

    V, Wm = pl.pallas_call(
        _globals_kernel,
        grid=(_B,),
        in_specs=[
            tok_spec, tok_spec, tok_spec,
            const_spec((_C, _C)), const_spec((1, _C)), const_spec((_C, _C)),
            const_spec((256, 3 * _C)), const_spec((1, 256)),
            const_spec((1, 256)), const_spec((1, 256)),
            const_spec((64, 256)), const_spec((1, 64)),
            const_spec((3, 64)), const_spec((1, 3)),
        ],
        out_specs=[
            pl.BlockSpec((1, 8, _C), lambda b: (b, 0, 0)),
            pl.BlockSpec((1, 8, 128), lambda b: (b, 0, 0)),
        ],
        out_shape=[
            jax.ShapeDtypeStruct((_B, 8, _C), jnp.float32),
            jax.ShapeDtypeStruct((_B, 8, 128), jnp.float32),
        ],
    )(rgb, nir, tir, Wq, bq2, Wk, W1, b12, lng2, lnb2, W2, b22, W3, b32)

    masked, mask4 = pl.pallas_call(
        _score_mask_kernel,
        grid=(_B,),
        in_specs=[
            tok_spec, tok_spec, tok_spec,
            pl.BlockSpec((1, 8, _C), lambda b: (b, 0, 0)),
            pl.BlockSpec((1, 8, 128), lambda b: (b, 0, 0)),
        ],
        out_specs=[
            pl.BlockSpec((3, 1, _N, _C), lambda b: (0, b, 0, 0)),
            pl.BlockSpec((3, 1, 1, _N), lambda b: (0, b, 0, 0)),
        ],
        out_shape=[
            jax.ShapeDtypeStruct((3, _B, _N, _C), jnp.float32),
            jax.ShapeDtypeStruct((3, _B, 1, _N), jnp.float32),
        ],
    )(rgb, nir, tir, V, Wm)

    return masked, mask4.reshape(3, _B, _N)


# speedup vs baseline: 3.3681x; 1.0704x over previous
import functools

import jax
import jax.numpy as jnp
from jax.experimental import pallas as pl

_B = 32
_N = 576
_C = 768
_SOFT_MASK_TAU = 0.3
_COSINE_TAU = 0.3
_SCALE = _C ** (-0.5)
_K_ORD = 230
_BISECT_ITERS = 24


def _gelu_exact(x):
    return 0.5 * x * (1.0 + jax.lax.erf(x * (2.0 ** -0.5)))


def _dot_t(a, b):
    return jax.lax.dot_general(a, b, (((1,), (1,)), ((), ())),
                               preferred_element_type=jnp.float32)


def _fused_kernel(rgb_ref, nir_ref, tir_ref, wq_ref, bq_ref, wk_ref,
                  w1_ref, b1_ref, lng_ref, lnb_ref, w2_ref, b2_ref,
                  w3_ref, b3_ref, out_ref, mask_ref):
    t_all = (rgb_ref[0], nir_ref[0], tir_ref[0])
    ones_n = jnp.ones((1, _N), jnp.float32)

    g_rows = [jax.lax.dot_general(ones_n, t, (((1,), (0,)), ((), ())),
                                  preferred_element_type=jnp.float32)
              * (1.0 / _N) for t in t_all]
    g = jnp.concatenate(g_rows, axis=0)
    gn = jnp.sqrt(jnp.sum(g * g, axis=1, keepdims=True))
    gh = g / (gn + 1e-8)

    q = _dot_t(g, wq_ref[...]) + bq_ref[...]
    qk = jax.lax.dot_general(q, wk_ref[...], (((1,), (0,)), ((), ())),
                             preferred_element_type=jnp.float32)

    g0, g1, g2 = g_rows
    cat = jnp.concatenate([
        jnp.concatenate([g0, g1, g2], axis=1),
        jnp.concatenate([g1, g0, g2], axis=1),
        jnp.concatenate([g2, g0, g1], axis=1),
    ], axis=0)
    h = _dot_t(cat, w1_ref[...]) + b1_ref[...]
    mu = jnp.mean(h, axis=1, keepdims=True)
    var = jnp.mean((h - mu) * (h - mu), axis=1, keepdims=True)
    h = (h - mu) / jnp.sqrt(var + 1e-5) * lng_ref[...] + lnb_ref[...]
    h = _gelu_exact(h)
    h = _gelu_exact(_dot_t(h, w2_ref[...]) + b2_ref[...])
    logits_w = _dot_t(h, w3_ref[...]) + b3_ref[...]
    lmax = jnp.max(logits_w, axis=1, keepdims=True)
    e = jnp.exp(logits_w - lmax)
    wmat = e / jnp.sum(e, axis=1, keepdims=True)

    v = jnp.concatenate([gh, qk], axis=0)

    scores = []
    for m in range(3):
        t = t_all[m]
        dots = _dot_t(v, t)
        n2 = _dot_t(jnp.ones((1, _C), jnp.float32), t * t)
        tnorm = jnp.sqrt(n2)
        cos = dots[0:3] / (tnorm + 1e-8)
        logits = dots[3:6] * _SCALE + cos * (1.0 / _COSINE_TAU)
        lm = jnp.max(logits, axis=1, keepdims=True)
        ex = jnp.exp(logits - lm)
        s = ex / jnp.sum(ex, axis=1, keepdims=True)
        mean = jnp.mean(s, axis=1, keepdims=True)
        d = s - mean
        sd = jnp.sqrt(jnp.sum(d * d, axis=1, keepdims=True) / (_N - 1)) + 1e-5
        ns = jax.nn.sigmoid(d / sd)
        j0, j1, j2 = ((0, 1, 2), (1, 0, 2), (2, 0, 1))[m]
        score = (ns[j0:j0 + 1] * wmat[m:m + 1, 0:1]
                 + ns[j1:j1 + 1] * wmat[m:m + 1, 1:2]
                 + ns[j2:j2 + 1] * wmat[m:m + 1, 2:3])
        scores.append(score)

    sc = jnp.concatenate(scores, axis=0)

    lo = jnp.zeros((3, 1), jnp.float32)
    hi = jnp.ones((3, 1), jnp.float32)
    kf = float(_K_ORD)
    for _ in range(_BISECT_ITERS):
        mid = 0.5 * (lo + hi)
        cnt = jnp.sum((sc < mid).astype(jnp.float32), axis=1, keepdims=True)
        below = cnt <= kf
        lo = jnp.where(below, mid, lo)
        hi = jnp.where(below, hi, mid)
    thr = jnp.max(jnp.where(sc < hi, sc, -jnp.inf), axis=1, keepdims=True)

    mask = jax.nn.sigmoid((sc - thr) * (1.0 / _SOFT_MASK_TAU))
    mask_ref[...] = mask.reshape(3, 1, 1, _N)
    for m in range(3):
        mcol = jnp.transpose(mask[m:m + 1])
        out_ref[m, 0] = t_all[m] * mcol


@functools.partial(jax.jit)
def kernel(rgb, nir, tir, Wq, bq, Wk, bk, W1, b1, ln_g, ln_b, W2, b2, W3, b3):
    del bk
    bq2 = bq.reshape(1, _C)
    b12 = b1.reshape(1, 256)
    lng2 = ln_g.reshape(1, 256)
    lnb2 = ln_b.reshape(1, 256)
    b22 = b2.reshape(1, 64)
    b32 = b3.reshape(1, 3)

    tok_spec = pl.BlockSpec((1, _N, _C), lambda b: (b, 0, 0))

    def const_spec(shape):
        nd = len(shape)
        return pl.BlockSpec(shape, lambda b, _nd=nd: (0,) * _nd)

    masked, mask4 = pl.pallas_call(
        _fused_kernel,
        grid=(_B,),
        in_specs=[
            tok_spec, tok_spec, tok_spec,
            const_spec((_C, _C)), const_spec((1, _C)), const_spec((_C, _C)),
            const_spec((256, 3 * _C)), const_spec((1, 256)),
            const_spec((1, 256)), const_spec((1, 256)),
            const_spec((64, 256)), const_spec((1, 64)),
            const_spec((3, 64)), const_spec((1, 3)),
        ],
        out_specs=[
            pl.BlockSpec((3, 1, _N, _C), lambda b: (0, b, 0, 0)),
            pl.BlockSpec((3, 1, 1, _N), lambda b: (0, b, 0, 0)),
        ],
        out_shape=[
            jax.ShapeDtypeStruct((3, _B, _N, _C), jnp.float32),
            jax.ShapeDtypeStruct((3, _B, 1, _N), jnp.float32),
        ],
    )(rgb, nir, tir, Wq, bq2, Wk, W1, b12, lng2, lnb2, W2, b22, W3, b32)

    return masked, mask4.reshape(3, _B, _N)


# speedup vs baseline: 4.3425x; 1.2893x over previous
import functools

import jax
import jax.numpy as jnp
from jax.experimental import pallas as pl

_B = 32
_N = 576
_C = 768
_NB = 2
_SOFT_MASK_TAU = 0.3
_COSINE_TAU = 0.3
_SCALE = _C ** (-0.5)
_K_ORD = 230
_BISECT_ITERS = 16


def _gelu_exact(x):
    return 0.5 * x * (1.0 + jax.lax.erf(x * (2.0 ** -0.5)))


def _dot_t(a, b):
    return jax.lax.dot_general(a, b, (((1,), (1,)), ((), ())),
                               preferred_element_type=jnp.float32)


def _batch_mask(t_all, wq, bq, wk, w1, b1, lng, lnb, w2, b2, w3, b3):
    ones_n = jnp.ones((1, _N), jnp.float32)
    g_rows = [jax.lax.dot_general(ones_n, t, (((1,), (0,)), ((), ())),
                                  preferred_element_type=jnp.float32)
              * (1.0 / _N) for t in t_all]
    g = jnp.concatenate(g_rows, axis=0)
    gn = jnp.sqrt(jnp.sum(g * g, axis=1, keepdims=True))
    gh = g / (gn + 1e-8)

    q = _dot_t(g, wq) + bq
    qk = jax.lax.dot_general(q, wk, (((1,), (0,)), ((), ())),
                             preferred_element_type=jnp.float32)

    g0, g1, g2 = g_rows
    cat = jnp.concatenate([
        jnp.concatenate([g0, g1, g2], axis=1),
        jnp.concatenate([g1, g0, g2], axis=1),
        jnp.concatenate([g2, g0, g1], axis=1),
    ], axis=0)
    h = _dot_t(cat, w1) + b1
    mu = jnp.mean(h, axis=1, keepdims=True)
    var = jnp.mean((h - mu) * (h - mu), axis=1, keepdims=True)
    h = (h - mu) / jnp.sqrt(var + 1e-5) * lng + lnb
    h = _gelu_exact(h)
    h = _gelu_exact(_dot_t(h, w2) + b2)
    logits_w = _dot_t(h, w3) + b3
    lmax = jnp.max(logits_w, axis=1, keepdims=True)
    e = jnp.exp(logits_w - lmax)
    wmat = e / jnp.sum(e, axis=1, keepdims=True)

    v = jnp.concatenate([gh, qk], axis=0)

    scores = []
    for m in range(3):
        t = t_all[m]
        dots = _dot_t(v, t)
        n2 = _dot_t(jnp.ones((1, _C), jnp.float32), t * t)
        tnorm = jnp.sqrt(n2)
        cos = dots[0:3] / (tnorm + 1e-8)
        logits = dots[3:6] * _SCALE + cos * (1.0 / _COSINE_TAU)
        lm = jnp.max(logits, axis=1, keepdims=True)
        ex = jnp.exp(logits - lm)
        s = ex / jnp.sum(ex, axis=1, keepdims=True)
        mean = jnp.mean(s, axis=1, keepdims=True)
        d = s - mean
        sd = jnp.sqrt(jnp.sum(d * d, axis=1, keepdims=True) / (_N - 1)) + 1e-5
        ns = jax.nn.sigmoid(d / sd)
        j0, j1, j2 = ((0, 1, 2), (1, 0, 2), (2, 0, 1))[m]
        score = (ns[j0:j0 + 1] * wmat[m:m + 1, 0:1]
                 + ns[j1:j1 + 1] * wmat[m:m + 1, 1:2]
                 + ns[j2:j2 + 1] * wmat[m:m + 1, 2:3])
        scores.append(score)

    sc = jnp.concatenate(scores, axis=0)

    lo = jnp.zeros((3, 1), jnp.float32)
    hi = jnp.ones((3, 1), jnp.float32)
    kf = float(_K_ORD)
    for _ in range(_BISECT_ITERS):
        mid = 0.5 * (lo + hi)
        cnt = jnp.sum((sc < mid).astype(jnp.float32), axis=1, keepdims=True)
        below = cnt <= kf
        lo = jnp.where(below, mid, lo)
        hi = jnp.where(below, hi, mid)
    thr = jnp.max(jnp.where(sc < hi, sc, -jnp.inf), axis=1, keepdims=True)

    return jax.nn.sigmoid((sc - thr) * (1.0 / _SOFT_MASK_TAU))


def _fused_kernel(rgb_ref, nir_ref, tir_ref, wq_ref, bq_ref, wk_ref,
                  w1_ref, b1_ref, lng_ref, lnb_ref, w2_ref, b2_ref,
                  w3_ref, b3_ref, out_ref, mask_ref):
    for bb in range(_NB):
        t_all = (rgb_ref[bb], nir_ref[bb], tir_ref[bb])
        mask = _batch_mask(t_all, wq_ref[...], bq_ref[...], wk_ref[...],
                           w1_ref[...], b1_ref[...], lng_ref[...],
                           lnb_ref[...], w2_ref[...], b2_ref[...],
                           w3_ref[...], b3_ref[...])
        mask_ref[:, bb] = mask.reshape(3, 1, _N)
        for m in range(3):
            mcol = jnp.transpose(mask[m:m + 1])
            out_ref[m, bb] = t_all[m] * mcol


@functools.partial(jax.jit)
def kernel(rgb, nir, tir, Wq, bq, Wk, bk, W1, b1, ln_g, ln_b, W2, b2, W3, b3):
    del bk
    bq2 = bq.reshape(1, _C)
    b12 = b1.reshape(1, 256)
    lng2 = ln_g.reshape(1, 256)
    lnb2 = ln_b.reshape(1, 256)
    b22 = b2.reshape(1, 64)
    b32 = b3.reshape(1, 3)

    tok_spec = pl.BlockSpec((_NB, _N, _C), lambda b: (b, 0, 0))

    def const_spec(shape):
        nd = len(shape)
        return pl.BlockSpec(shape, lambda b, _nd=nd: (0,) * _nd)

    masked, mask4 = pl.pallas_call(
        _fused_kernel,
        grid=(_B // _NB,),
        in_specs=[
            tok_spec, tok_spec, tok_spec,
            const_spec((_C, _C)), const_spec((1, _C)), const_spec((_C, _C)),
            const_spec((256, 3 * _C)), const_spec((1, 256)),
            const_spec((1, 256)), const_spec((1, 256)),
            const_spec((64, 256)), const_spec((1, 64)),
            const_spec((3, 64)), const_spec((1, 3)),
        ],
        out_specs=[
            pl.BlockSpec((3, _NB, _N, _C), lambda b: (0, b, 0, 0)),
            pl.BlockSpec((3, _NB, 1, _N), lambda b: (0, b, 0, 0)),
        ],
        out_shape=[
            jax.ShapeDtypeStruct((3, _B, _N, _C), jnp.float32),
            jax.ShapeDtypeStruct((3, _B, 1, _N), jnp.float32),
        ],
    )(rgb, nir, tir, Wq, bq2, Wk, W1, b12, lng2, lnb2, W2, b22, W3, b32)

    return masked, mask4.reshape(3, _B, _N)


# speedup vs baseline: 5.2889x; 1.2179x over previous
import functools

import jax
import jax.numpy as jnp
from jax.experimental import pallas as pl

_B = 32
_N = 576
_C = 768
_NB = 2
_SOFT_MASK_TAU = 0.3
_COSINE_TAU = 0.3
_SCALE = _C ** (-0.5)
_K_ORD = 230
_BISECT_ITERS = 16


def _gelu_exact(x):
    return 0.5 * x * (1.0 + jax.lax.erf(x * (2.0 ** -0.5)))


def _dot_t(a, b):
    return jax.lax.dot_general(a, b, (((1,), (1,)), ((), ())),
                               preferred_element_type=jnp.float32)


def _fused_kernel(rgb_ref, nir_ref, tir_ref, wq_ref, bq_ref, wk_ref,
                  w1_ref, b1_ref, lng_ref, lnb_ref, w2_ref, b2_ref,
                  w3_ref, b3_ref, out_ref, mask_ref):
    ones_n = jnp.ones((1, _N), jnp.float32)
    ones_c = jnp.ones((1, _C), jnp.float32)
    blocks = [(rgb_ref[bb], nir_ref[bb], tir_ref[bb]) for bb in range(_NB)]

    g_rows = [jax.lax.dot_general(ones_n, t, (((1,), (0,)), ((), ())),
                                  preferred_element_type=jnp.float32)
              * (1.0 / _N)
              for t_all in blocks for t in t_all]
    g = jnp.concatenate(g_rows, axis=0)
    gn = jnp.sqrt(jnp.sum(g * g, axis=1, keepdims=True))
    gh = g / (gn + 1e-8)

    q = _dot_t(g, wq_ref[...]) + bq_ref[...]
    qk = jax.lax.dot_general(q, wk_ref[...], (((1,), (0,)), ((), ())),
                             preferred_element_type=jnp.float32)

    cats = []
    for bb in range(_NB):
        g0, g1, g2 = g_rows[3 * bb:3 * bb + 3]
        cats += [
            jnp.concatenate([g0, g1, g2], axis=1),
            jnp.concatenate([g1, g0, g2], axis=1),
            jnp.concatenate([g2, g0, g1], axis=1),
        ]
    cat = jnp.concatenate(cats, axis=0)
    h = _dot_t(cat, w1_ref[...]) + b1_ref[...]
    mu = jnp.mean(h, axis=1, keepdims=True)
    var = jnp.mean((h - mu) * (h - mu), axis=1, keepdims=True)
    h = (h - mu) / jnp.sqrt(var + 1e-5) * lng_ref[...] + lnb_ref[...]
    h = _gelu_exact(h)
    h = _gelu_exact(_dot_t(h, w2_ref[...]) + b2_ref[...])
    logits_w = _dot_t(h, w3_ref[...]) + b3_ref[...]
    lmax = jnp.max(logits_w, axis=1, keepdims=True)
    e = jnp.exp(logits_w - lmax)
    wmat = e / jnp.sum(e, axis=1, keepdims=True)

    scores = []
    for bb in range(_NB):
        v = jnp.concatenate([gh[3 * bb:3 * bb + 3], qk[3 * bb:3 * bb + 3]],
                            axis=0)
        for m in range(3):
            t = blocks[bb][m]
            dots = _dot_t(v, t)
            n2 = _dot_t(ones_c, t * t)
            tnorm = jnp.sqrt(n2)
            cos = dots[0:3] / (tnorm + 1e-8)
            logits = dots[3:6] * _SCALE + cos * (1.0 / _COSINE_TAU)
            lm = jnp.max(logits, axis=1, keepdims=True)
            ex = jnp.exp(logits - lm)
            s = ex / jnp.sum(ex, axis=1, keepdims=True)
            d = s - (1.0 / _N)
            sd = (jnp.sqrt(jnp.sum(d * d, axis=1, keepdims=True) / (_N - 1))
                  + 1e-5)
            ns = jax.nn.sigmoid(d / sd)
            j0, j1, j2 = ((0, 1, 2), (1, 0, 2), (2, 0, 1))[m]
            r = 3 * bb + m
            scores.append(ns[j0:j0 + 1] * wmat[r:r + 1, 0:1]
                          + ns[j1:j1 + 1] * wmat[r:r + 1, 1:2]
                          + ns[j2:j2 + 1] * wmat[r:r + 1, 2:3])

    sc = jnp.concatenate(scores, axis=0)

    lo = jnp.zeros((3 * _NB, 1), jnp.float32)
    hi = jnp.ones((3 * _NB, 1), jnp.float32)
    kf = float(_K_ORD)
    for _ in range(_BISECT_ITERS):
        mid = 0.5 * (lo + hi)
        cnt = jnp.sum((sc < mid).astype(jnp.float32), axis=1, keepdims=True)
        below = cnt <= kf
        lo = jnp.where(below, mid, lo)
        hi = jnp.where(below, hi, mid)
    thr = jnp.max(jnp.where(sc < hi, sc, -jnp.inf), axis=1, keepdims=True)
    mask = jax.nn.sigmoid((sc - thr) * (1.0 / _SOFT_MASK_TAU))

    mask_ref[...] = mask.reshape(_NB, 3, 1, _N).transpose(1, 0, 2, 3)
    for bb in range(_NB):
        for m in range(3):
            r = 3 * bb + m
            mcol = jnp.transpose(mask[r:r + 1])
            out_ref[m, bb] = blocks[bb][m] * mcol


@functools.partial(jax.jit)
def kernel(rgb, nir, tir, Wq, bq, Wk, bk, W1, b1, ln_g, ln_b, W2, b2, W3, b3):
    del bk
    bq2 = bq.reshape(1, _C)
    b12 = b1.reshape(1, 256)
    lng2 = ln_g.reshape(1, 256)
    lnb2 = ln_b.reshape(1, 256)
    b22 = b2.reshape(1, 64)
    b32 = b3.reshape(1, 3)

    tok_spec = pl.BlockSpec((_NB, _N, _C), lambda b: (b, 0, 0))

    def const_spec(shape):
        nd = len(shape)
        return pl.BlockSpec(shape, lambda b, _nd=nd: (0,) * _nd)

    masked, mask4 = pl.pallas_call(
        _fused_kernel,
        grid=(_B // _NB,),
        in_specs=[
            tok_spec, tok_spec, tok_spec,
            const_spec((_C, _C)), const_spec((1, _C)), const_spec((_C, _C)),
            const_spec((256, 3 * _C)), const_spec((1, 256)),
            const_spec((1, 256)), const_spec((1, 256)),
            const_spec((64, 256)), const_spec((1, 64)),
            const_spec((3, 64)), const_spec((1, 3)),
        ],
        out_specs=[
            pl.BlockSpec((3, _NB, _N, _C), lambda b: (0, b, 0, 0)),
            pl.BlockSpec((3, _NB, 1, _N), lambda b: (0, b, 0, 0)),
        ],
        out_shape=[
            jax.ShapeDtypeStruct((3, _B, _N, _C), jnp.float32),
            jax.ShapeDtypeStruct((3, _B, 1, _N), jnp.float32),
        ],
    )(rgb, nir, tir, Wq, bq2, Wk, W1, b12, lng2, lnb2, W2, b22, W3, b32)

    return masked, mask4.reshape(3, _B, _N)


# speedup vs baseline: 6.1238x; 1.1579x over previous
import functools

import jax
import jax.numpy as jnp
from jax.experimental import pallas as pl

_B = 32
_N = 576
_C = 768
_NB = 2
_SOFT_MASK_TAU = 0.3
_COSINE_TAU = 0.3
_SCALE = _C ** (-0.5)
_K_ORD = 230
_BISECT_ITERS = 16


def _gelu_exact(x):
    return 0.5 * x * (1.0 + jax.lax.erf(x * (2.0 ** -0.5)))


def _dot_t(a, b):
    return jax.lax.dot_general(a, b, (((1,), (1,)), ((), ())),
                               preferred_element_type=jnp.float32)


def _fused_kernel(rgb_ref, nir_ref, tir_ref, wq_ref, bq_ref, wk_ref,
                  w1_ref, b1_ref, lng_ref, lnb_ref, w2_ref, b2_ref,
                  w3_ref, b3_ref, out_ref, mask_ref):
    ones_n = jnp.ones((1, _N), jnp.float32)
    blocks = [(rgb_ref[bb], nir_ref[bb], tir_ref[bb]) for bb in range(_NB)]

    g_rows = [jax.lax.dot_general(ones_n, t, (((1,), (0,)), ((), ())),
                                  preferred_element_type=jnp.float32)
              * (1.0 / _N)
              for t_all in blocks for t in t_all]
    g = jnp.concatenate(g_rows, axis=0)
    gn = jnp.sqrt(jnp.sum(g * g, axis=1, keepdims=True))
    gh = g / (gn + 1e-8)

    q = _dot_t(g, wq_ref[...]) + bq_ref[...]
    qk = jax.lax.dot_general(q, wk_ref[...], (((1,), (0,)), ((), ())),
                             preferred_element_type=jnp.float32)

    cats = []
    for bb in range(_NB):
        g0, g1, g2 = g_rows[3 * bb:3 * bb + 3]
        cats += [
            jnp.concatenate([g0, g1, g2], axis=1),
            jnp.concatenate([g1, g0, g2], axis=1),
            jnp.concatenate([g2, g0, g1], axis=1),
        ]
    cat = jnp.concatenate(cats, axis=0)
    h = _dot_t(cat, w1_ref[...]) + b1_ref[...]
    mu = jnp.mean(h, axis=1, keepdims=True)
    var = jnp.mean((h - mu) * (h - mu), axis=1, keepdims=True)
    h = (h - mu) / jnp.sqrt(var + 1e-5) * lng_ref[...] + lnb_ref[...]
    h = _gelu_exact(h)
    h = _gelu_exact(_dot_t(h, w2_ref[...]) + b2_ref[...])
    logits_w = _dot_t(h, w3_ref[...]) + b3_ref[...]
    lmax = jnp.max(logits_w, axis=1, keepdims=True)
    e = jnp.exp(logits_w - lmax)
    wmat = e / jnp.sum(e, axis=1, keepdims=True)

    scores = []
    for bb in range(_NB):
        v = jnp.concatenate([gh[3 * bb:3 * bb + 3], qk[3 * bb:3 * bb + 3]],
                            axis=0)
        for m in range(3):
            t = blocks[bb][m]
            dots = _dot_t(v, t)
            n2c = jnp.sum(t * t, axis=1, keepdims=True)
            tnorm = jnp.sqrt(jnp.transpose(n2c))
            cos = dots[0:3] / (tnorm + 1e-8)
            logits = dots[3:6] * _SCALE + cos * (1.0 / _COSINE_TAU)
            lm = jnp.max(logits, axis=1, keepdims=True)
            ex = jnp.exp(logits - lm)
            s = ex / jnp.sum(ex, axis=1, keepdims=True)
            d = s - (1.0 / _N)
            sd = (jnp.sqrt(jnp.sum(d * d, axis=1, keepdims=True) / (_N - 1))
                  + 1e-5)
            ns = jax.nn.sigmoid(d / sd)
            j0, j1, j2 = ((0, 1, 2), (1, 0, 2), (2, 0, 1))[m]
            r = 3 * bb + m
            scores.append(ns[j0:j0 + 1] * wmat[r:r + 1, 0:1]
                          + ns[j1:j1 + 1] * wmat[r:r + 1, 1:2]
                          + ns[j2:j2 + 1] * wmat[r:r + 1, 2:3])

    sc = jnp.concatenate(scores, axis=0)

    lo = jnp.zeros((3 * _NB, 1), jnp.float32)
    hi = jnp.ones((3 * _NB, 1), jnp.float32)
    kf = float(_K_ORD)
    for _ in range(_BISECT_ITERS):
        mid = 0.5 * (lo + hi)
        cnt = jnp.sum((sc < mid).astype(jnp.float32), axis=1, keepdims=True)
        below = cnt <= kf
        lo = jnp.where(below, mid, lo)
        hi = jnp.where(below, hi, mid)
    thr = jnp.max(jnp.where(sc < hi, sc, -jnp.inf), axis=1, keepdims=True)
    mask = jax.nn.sigmoid((sc - thr) * (1.0 / _SOFT_MASK_TAU))

    mask_cols = jnp.transpose(mask)
    for bb in range(_NB):
        mask_ref[:, bb] = mask[3 * bb:3 * bb + 3].reshape(3, 1, _N)
        for m in range(3):
            r = 3 * bb + m
            out_ref[m, bb] = blocks[bb][m] * mask_cols[:, r:r + 1]


@functools.partial(jax.jit)
def kernel(rgb, nir, tir, Wq, bq, Wk, bk, W1, b1, ln_g, ln_b, W2, b2, W3, b3):
    del bk
    bq2 = bq.reshape(1, _C)
    b12 = b1.reshape(1, 256)
    lng2 = ln_g.reshape(1, 256)
    lnb2 = ln_b.reshape(1, 256)
    b22 = b2.reshape(1, 64)
    b32 = b3.reshape(1, 3)

    tok_spec = pl.BlockSpec((_NB, _N, _C), lambda b: (b, 0, 0))

    def const_spec(shape):
        nd = len(shape)
        return pl.BlockSpec(shape, lambda b, _nd=nd: (0,) * _nd)

    masked, mask4 = pl.pallas_call(
        _fused_kernel,
        grid=(_B // _NB,),
        in_specs=[
            tok_spec, tok_spec, tok_spec,
            const_spec((_C, _C)), const_spec((1, _C)), const_spec((_C, _C)),
            const_spec((256, 3 * _C)), const_spec((1, 256)),
            const_spec((1, 256)), const_spec((1, 256)),
            const_spec((64, 256)), const_spec((1, 64)),
            const_spec((3, 64)), const_spec((1, 3)),
        ],
        out_specs=[
            pl.BlockSpec((3, _NB, _N, _C), lambda b: (0, b, 0, 0)),
            pl.BlockSpec((3, _NB, 1, _N), lambda b: (0, b, 0, 0)),
        ],
        out_shape=[
            jax.ShapeDtypeStruct((3, _B, _N, _C), jnp.float32),
            jax.ShapeDtypeStruct((3, _B, 1, _N), jnp.float32),
        ],
    )(rgb, nir, tir, Wq, bq2, Wk, W1, b12, lng2, lnb2, W2, b22, W3, b32)

    return masked, mask4.reshape(3, _B, _N)
